# baseline (speedup 1.0000x reference)
import jax
import jax.numpy as jnp
from jax.experimental import pallas as pl
from jax.experimental.pallas import tpu as pltpu

_B, _N2, _N1, _F, _A, _O, _L = 1024, 10, 25, 256, 128, 256, 50
_BB = 32


def _l2n(x):
    return x * jax.lax.rsqrt(jnp.maximum(jnp.sum(x * x, axis=-1, keepdims=True), 1e-12))


def _fused(x0_ref, x1_ref, x2_ref, wagg0_ref, w0s_ref, w0a_ref, wagg1_ref,
           w1s_ref, w1a_ref, wcls_ref, out_ref):
    bb = _BB
    x2 = x2_ref[...].reshape(bb * _N2 * _N1, _F)
    t = jnp.maximum(jnp.dot(x2, wagg0_ref[...], preferred_element_type=jnp.float32), 0.0)
    agg0 = jnp.mean(t.reshape(bb * _N2, _N1, _A), axis=1)
    x1 = x1_ref[...].reshape(bb * _N2, _F)
    h1 = jnp.maximum(
        jnp.dot(x1, w0s_ref[...], preferred_element_type=jnp.float32)
        + jnp.dot(agg0, w0a_ref[...], preferred_element_type=jnp.float32), 0.0)
    h1 = _l2n(h1)
    g = jnp.maximum(jnp.dot(h1, wagg1_ref[...], preferred_element_type=jnp.float32), 0.0)
    agg1 = jnp.mean(g.reshape(bb, _N2, _A), axis=1)
    h0 = (jnp.dot(x0_ref[...], w1s_ref[...], preferred_element_type=jnp.float32)
          + jnp.dot(agg1, w1a_ref[...], preferred_element_type=jnp.float32))
    h0 = _l2n(_l2n(h0))
    out_ref[...] = jnp.maximum(
        jnp.dot(h0, wcls_ref[...], preferred_element_type=jnp.float32), 0.0)


def _full(shape):
    return pl.BlockSpec(shape, lambda i: (0,) * len(shape))


def kernel(x0, x1, x2, Wagg0, Wagg1, Wcomb0, Wcomb1, Wcls):
    w0s, w0a = Wcomb0[:_F], Wcomb0[_F:]
    w1s, w1a = Wcomb1[:_F], Wcomb1[_F:]
    x2r = x2.reshape(_B, _N2 * _N1, _F)
    return pl.pallas_call(
        _fused,
        grid=(_B // _BB,),
        in_specs=[
            pl.BlockSpec((_BB, _F), lambda i: (i, 0)),
            pl.BlockSpec((_BB, _N2, _F), lambda i: (i, 0, 0)),
            pl.BlockSpec((_BB, _N2 * _N1, _F), lambda i: (i, 0, 0)),
            _full((_F, _A)), _full((_F, _O)), _full((_A, _O)),
            _full((_O, _A)), _full((_F, _O)), _full((_A, _O)),
            _full((_O, _L)),
        ],
        out_specs=pl.BlockSpec((_BB, _L), lambda i: (i, 0)),
        out_shape=jax.ShapeDtypeStruct((_B, _L), jnp.float32),
        compiler_params=pltpu.CompilerParams(dimension_semantics=("parallel",)),
    )(x0, x1, x2r, Wagg0, w0s, w0a, Wagg1, w1s, w1a, Wcls)
